# baseline (device time: 170570 ns/iter reference)
import jax
import jax.numpy as jnp
from jax import lax
from jax.experimental import pallas as pl
from jax.experimental.pallas import tpu as pltpu

N_DEV = 16
B, SQ, SKV = 2, 512, 512
HQ_PER = 8
DH = 64
D_MODEL = 768
BLK = 64
ROWS = B * SQ

COMM_DT = jnp.bfloat16
F32 = jnp.float32

RS_LEN = [ROWS >> (k + 1) for k in range(4)]
RS_OFF = [0, 512, 768, 896]
AG_LEN = {k: ROWS >> (k + 1) for k in range(4)}
AG_OFF = {3: 0, 2: 64, 1: 192, 0: 448}
MESH_ID = pl.DeviceIdType.MESH


def kernel(x, Wq, K_ext, V_ext, Wo):
    def body(x_ref, wq_ref, k_hbm, v_hbm, wo_ref, out_ref,
             k_ref, v_ref, acc_ref, stage_ref, rs_bufs, ag_bufs,
             copy_sems, send_sems, recv_sems):
        p = lax.axis_index("i")
        z = p // 4
        b4 = p % 4
        by = b4 // 2
        bx = (b4 % 2) ^ by
        bz0 = z % 2
        bz1 = z // 2
        bits = [bx, by, bz0, bz1]

        def ring_pos(xx, yy, zz):
            return zz * 4 + yy * 2 + (xx ^ yy)

        partners = [
            ring_pos(1 - bx, by, z),
            ring_pos(bx, 1 - by, z),
            ring_pos(bx, by, 2 * bz1 + (1 - bz0)),
            ring_pos(bx, by, 2 * (1 - bz1) + bz0),
        ]

        NSPLIT = 4
        SSEG = SKV // NSPLIT
        copies = []
        for i, (hbm, vmem) in enumerate(((k_hbm, k_ref), (v_hbm, v_ref))):
            for b in range(B):
                for si in range(NSPLIT):
                    c = pltpu.make_async_copy(
                        hbm.at[b, pl.ds(si * SSEG, SSEG),
                               pl.ds(p * HQ_PER, HQ_PER), :],
                        vmem.at[b, pl.ds(si * SSEG, SSEG), :, :],
                        copy_sems.at[i * B * NSPLIT + b * NSPLIT + si])
                    c.start()
                    copies.append(c)

        barrier = pltpu.get_barrier_semaphore()
        for pr in partners:
            pl.semaphore_signal(barrier, inc=1, device_id=(pr,),
                                device_id_type=MESH_ID)
        pl.semaphore_wait(barrier, 4)

        qb = lax.broadcasted_iota(jnp.int32, (SQ, SKV), 0) // BLK
        kb = lax.broadcasted_iota(jnp.int32, (SQ, SKV), 1) // BLK
        mask = kb <= qb
        wq16 = wq_ref[...].astype(COMM_DT)
        wo16 = wo_ref[...].astype(COMM_DT)
        for b in range(B):
            x16 = x_ref[b].astype(COMM_DT)
            Qb = jnp.dot(x16, wq16, preferred_element_type=F32)
            if b == 0:
                for c in copies:
                    c.wait()
            part = jnp.zeros((SQ, D_MODEL), dtype=F32)
            for h in range(HQ_PER):
                q = Qb[:, h * DH:(h + 1) * DH].astype(COMM_DT)
                k = k_ref[b, :, h, :].astype(COMM_DT)
                v = v_ref[b, :, h, :].astype(COMM_DT)
                s = lax.dot_general(
                    q, k, (((1,), (1,)), ((), ())),
                    preferred_element_type=F32) * 0.125
                s = jnp.where(mask, s, -1e9)
                m = jnp.max(s, axis=1, keepdims=True)
                w = jnp.exp(s - m)
                w = (w / jnp.sum(w, axis=1, keepdims=True)).astype(COMM_DT)
                ctx_h = jnp.dot(w, v, preferred_element_type=F32)
                part = part + jnp.dot(
                    ctx_h.astype(COMM_DT), wo16[h * DH:(h + 1) * DH, :],
                    preferred_element_type=F32)
            acc_ref[pl.ds(b * SQ, SQ), :] = part

        seg = p * 0
        for k in range(4):
            half = RS_LEN[k]
            bit = bits[k]
            send_start = seg + (1 - bit) * half
            keep_start = seg + bit * half
            stage_ref[pl.ds(0, half), :] = (
                acc_ref[pl.ds(send_start, half), :].astype(COMM_DT))
            rdma = pltpu.make_async_remote_copy(
                src_ref=stage_ref.at[pl.ds(0, half), :],
                dst_ref=rs_bufs.at[pl.ds(RS_OFF[k], half), :],
                send_sem=send_sems.at[k],
                recv_sem=recv_sems.at[k],
                device_id=(partners[k],),
                device_id_type=MESH_ID,
            )
            rdma.start()
            rdma.wait()
            sl = pl.ds(keep_start, half)
            acc_ref[sl, :] = acc_ref[sl, :] + (
                rs_bufs[pl.ds(RS_OFF[k], half), :].astype(F32))
            seg = keep_start

        for k in (3, 2, 1, 0):
            L = AG_LEN[k]
            off = AG_OFF[k]
            bit = bits[k]
            partner_start = seg + (1 - 2 * bit) * L
            stage_ref[pl.ds(0, L), :] = (
                acc_ref[pl.ds(seg, L), :].astype(COMM_DT))
            rdma = pltpu.make_async_remote_copy(
                src_ref=stage_ref.at[pl.ds(0, L), :],
                dst_ref=ag_bufs.at[pl.ds(off, L), :],
                send_sem=send_sems.at[7 - k],
                recv_sem=recv_sems.at[7 - k],
                device_id=(partners[k],),
                device_id_type=MESH_ID,
            )
            rdma.start()
            rdma.wait()
            acc_ref[pl.ds(partner_start, L), :] = (
                ag_bufs[pl.ds(off, L), :].astype(F32))
            seg = seg - bit * L

        out_ref[0, :, :] = acc_ref[0:SQ, :]
        out_ref[1, :, :] = acc_ref[SQ:ROWS, :]

    return pl.pallas_call(
        body,
        out_shape=jax.ShapeDtypeStruct((B, SQ, D_MODEL), F32),
        in_specs=[
            pl.BlockSpec(memory_space=pltpu.VMEM),
            pl.BlockSpec(memory_space=pltpu.VMEM),
            pl.BlockSpec(memory_space=pl.ANY),
            pl.BlockSpec(memory_space=pl.ANY),
            pl.BlockSpec(memory_space=pltpu.VMEM),
        ],
        out_specs=pl.BlockSpec(memory_space=pltpu.VMEM),
        scratch_shapes=[
            pltpu.VMEM((B, SKV, HQ_PER, DH), F32),
            pltpu.VMEM((B, SKV, HQ_PER, DH), F32),
            pltpu.VMEM((ROWS, D_MODEL), F32),
            pltpu.VMEM((512, D_MODEL), COMM_DT),
            pltpu.VMEM((960, D_MODEL), COMM_DT),
            pltpu.VMEM((960, D_MODEL), COMM_DT),
            pltpu.SemaphoreType.DMA((16,)),
            pltpu.SemaphoreType.DMA((8,)),
            pltpu.SemaphoreType.DMA((8,)),
        ],
        compiler_params=pltpu.CompilerParams(collective_id=0),
    )(x, Wq, K_ext, V_ext, Wo)


# device time: 115513 ns/iter; 1.4766x vs baseline; 1.4766x over previous
import jax
import jax.numpy as jnp
from jax import lax
from jax.experimental import pallas as pl
from jax.experimental.pallas import tpu as pltpu

N_DEV = 16
B, SQ, SKV = 2, 512, 512
HQ_PER = 8
DH = 64
D_MODEL = 768
BLK = 64
ROWS = B * SQ

COMM_DT = jnp.bfloat16
F32 = jnp.float32

RS_LEN = [ROWS >> (k + 1) for k in range(4)]
RS_OFF = [0, 512, 768, 896]
AG_LEN = {k: ROWS >> (k + 1) for k in range(4)}
AG_OFF = {3: 0, 2: 64, 1: 192, 0: 448}
MESH_ID = pl.DeviceIdType.MESH


def kernel(x, Wq, K_ext, V_ext, Wo):
    my = lax.axis_index("i")
    K_loc = lax.dynamic_slice_in_dim(
        K_ext, my * HQ_PER, HQ_PER, axis=2).astype(COMM_DT)
    V_loc = lax.dynamic_slice_in_dim(
        V_ext, my * HQ_PER, HQ_PER, axis=2).astype(COMM_DT)

    def body(x_ref, wq_ref, k_ref, v_ref, wo_ref, out_ref,
             acc_ref, stage_ref, rs_bufs, ag_bufs, send_sems, recv_sems):
        p = lax.axis_index("i")
        z = p // 4
        b4 = p % 4
        by = b4 // 2
        bx = (b4 % 2) ^ by
        bz0 = z % 2
        bz1 = z // 2
        bits = [bx, by, bz0, bz1]

        def ring_pos(xx, yy, zz):
            return zz * 4 + yy * 2 + (xx ^ yy)

        partners = [
            ring_pos(1 - bx, by, z),
            ring_pos(bx, 1 - by, z),
            ring_pos(bx, by, 2 * bz1 + (1 - bz0)),
            ring_pos(bx, by, 2 * (1 - bz1) + bz0),
        ]

        barrier = pltpu.get_barrier_semaphore()
        for pr in partners:
            pl.semaphore_signal(barrier, inc=1, device_id=(pr,),
                                device_id_type=MESH_ID)
        pl.semaphore_wait(barrier, 4)

        qb = lax.broadcasted_iota(jnp.int32, (SQ, SKV), 0) // BLK
        kb = lax.broadcasted_iota(jnp.int32, (SQ, SKV), 1) // BLK
        mask = kb <= qb
        wq16 = wq_ref[...].astype(COMM_DT)
        wo16 = wo_ref[...].astype(COMM_DT)

        def compute_batch(b):
            x16 = x_ref[b].astype(COMM_DT)
            Qb = jnp.dot(x16, wq16, preferred_element_type=F32)
            part = jnp.zeros((SQ, D_MODEL), dtype=F32)
            for h in range(HQ_PER):
                q = Qb[:, h * DH:(h + 1) * DH].astype(COMM_DT)
                k = k_ref[b, :, h, :]
                v = v_ref[b, :, h, :]
                s = lax.dot_general(
                    q, k, (((1,), (1,)), ((), ())),
                    preferred_element_type=F32) * 0.125
                s = jnp.where(mask, s, -1e9)
                m = jnp.max(s, axis=1, keepdims=True)
                w = jnp.exp(s - m)
                w = (w / jnp.sum(w, axis=1, keepdims=True)).astype(COMM_DT)
                ctx_h = jnp.dot(w, v, preferred_element_type=F32)
                part = part + jnp.dot(
                    ctx_h.astype(COMM_DT), wo16[h * DH:(h + 1) * DH, :],
                    preferred_element_type=F32)
            acc_ref[pl.ds(b * SQ, SQ), :] = part

        compute_batch(1 - bx)

        seg = p * 0
        rdma0 = None
        for k in range(4):
            half = RS_LEN[k]
            bit = bits[k]
            send_start = seg + (1 - bit) * half
            keep_start = seg + bit * half
            stage_ref[pl.ds(0, half), :] = (
                acc_ref[pl.ds(send_start, half), :].astype(COMM_DT))
            rdma = pltpu.make_async_remote_copy(
                src_ref=stage_ref.at[pl.ds(0, half), :],
                dst_ref=rs_bufs.at[pl.ds(RS_OFF[k], half), :],
                send_sem=send_sems.at[k],
                recv_sem=recv_sems.at[k],
                device_id=(partners[k],),
                device_id_type=MESH_ID,
            )
            rdma.start()
            if k == 0:
                compute_batch(bx)
            rdma.wait()
            sl = pl.ds(keep_start, half)
            acc_ref[sl, :] = acc_ref[sl, :] + (
                rs_bufs[pl.ds(RS_OFF[k], half), :].astype(F32))
            seg = keep_start

        for k in (3, 2, 1, 0):
            L = AG_LEN[k]
            off = AG_OFF[k]
            bit = bits[k]
            partner_start = seg + (1 - 2 * bit) * L
            stage_ref[pl.ds(0, L), :] = (
                acc_ref[pl.ds(seg, L), :].astype(COMM_DT))
            rdma = pltpu.make_async_remote_copy(
                src_ref=stage_ref.at[pl.ds(0, L), :],
                dst_ref=ag_bufs.at[pl.ds(off, L), :],
                send_sem=send_sems.at[7 - k],
                recv_sem=recv_sems.at[7 - k],
                device_id=(partners[k],),
                device_id_type=MESH_ID,
            )
            rdma.start()
            rdma.wait()
            acc_ref[pl.ds(partner_start, L), :] = (
                ag_bufs[pl.ds(off, L), :].astype(F32))
            seg = seg - bit * L

        out_ref[0, :, :] = acc_ref[0:SQ, :]
        out_ref[1, :, :] = acc_ref[SQ:ROWS, :]

    return pl.pallas_call(
        body,
        out_shape=jax.ShapeDtypeStruct((B, SQ, D_MODEL), F32),
        in_specs=[pl.BlockSpec(memory_space=pltpu.VMEM)] * 5,
        out_specs=pl.BlockSpec(memory_space=pltpu.VMEM),
        scratch_shapes=[
            pltpu.VMEM((ROWS, D_MODEL), F32),
            pltpu.VMEM((512, D_MODEL), COMM_DT),
            pltpu.VMEM((960, D_MODEL), COMM_DT),
            pltpu.VMEM((960, D_MODEL), COMM_DT),
            pltpu.SemaphoreType.DMA((8,)),
            pltpu.SemaphoreType.DMA((8,)),
        ],
        compiler_params=pltpu.CompilerParams(collective_id=0),
    )(x, Wq, K_loc, V_loc, Wo)


# device time: 112497 ns/iter; 1.5162x vs baseline; 1.0268x over previous
import jax
import jax.numpy as jnp
from jax import lax
from jax.experimental import pallas as pl
from jax.experimental.pallas import tpu as pltpu

N_DEV = 16
B, SQ, SKV = 2, 512, 512
HQ_PER = 8
DH = 64
D_MODEL = 768
BLK = 64
ROWS = B * SQ

COMM_DT = jnp.bfloat16
F32 = jnp.float32

RS_LEN = [ROWS >> (k + 1) for k in range(4)]
RS_OFF = [0, 512, 768, 896]
AG_LEN = {k: ROWS >> (k + 1) for k in range(4)}
AG_OFF = {3: 0, 2: 64, 1: 192, 0: 448}
MESH_ID = pl.DeviceIdType.MESH


def kernel(x, Wq, K_ext, V_ext, Wo):
    my = lax.axis_index("i")
    K_loc = jnp.transpose(
        lax.dynamic_slice_in_dim(K_ext, my * HQ_PER, HQ_PER, axis=2),
        (0, 2, 1, 3)).astype(COMM_DT)
    V_loc = jnp.transpose(
        lax.dynamic_slice_in_dim(V_ext, my * HQ_PER, HQ_PER, axis=2),
        (0, 2, 1, 3)).astype(COMM_DT)

    def body(x_ref, wq_ref, k_ref, v_ref, wo_ref, out_ref,
             acc_ref, stage_ref, rs_bufs, ag_bufs, send_sems, recv_sems):
        p = lax.axis_index("i")
        z = p // 4
        b4 = p % 4
        by = b4 // 2
        bx = (b4 % 2) ^ by
        bz0 = z % 2
        bz1 = z // 2
        bits = [bx, by, bz0, bz1]

        def ring_pos(xx, yy, zz):
            return zz * 4 + yy * 2 + (xx ^ yy)

        partners = [
            ring_pos(1 - bx, by, z),
            ring_pos(bx, 1 - by, z),
            ring_pos(bx, by, 2 * bz1 + (1 - bz0)),
            ring_pos(bx, by, 2 * (1 - bz1) + bz0),
        ]

        barrier = pltpu.get_barrier_semaphore()
        for pr in partners:
            pl.semaphore_signal(barrier, inc=1, device_id=(pr,),
                                device_id_type=MESH_ID)
        pl.semaphore_wait(barrier, 4)

        qb = lax.broadcasted_iota(jnp.int32, (SQ, SKV), 0) // BLK
        kb = lax.broadcasted_iota(jnp.int32, (SQ, SKV), 1) // BLK
        mask = kb <= qb
        wq16 = wq_ref[...].astype(COMM_DT)
        wo16 = wo_ref[...].astype(COMM_DT)

        def compute_batch(b):
            x16 = x_ref[b].astype(COMM_DT)
            Qb = jnp.dot(x16, wq16, preferred_element_type=F32)
            part = jnp.zeros((SQ, D_MODEL), dtype=F32)
            for h in range(HQ_PER):
                q = Qb[:, h * DH:(h + 1) * DH].astype(COMM_DT)
                k = k_ref[b, h, :, :]
                v = v_ref[b, h, :, :]
                s = lax.dot_general(
                    q, k, (((1,), (1,)), ((), ())),
                    preferred_element_type=F32) * 0.125
                s = jnp.where(mask, s, -1e9)
                m = jnp.max(s, axis=1, keepdims=True)
                w = jnp.exp(s - m)
                ctx_h = jnp.dot(w.astype(COMM_DT), v,
                                preferred_element_type=F32)
                ctx_h = ctx_h / jnp.sum(w, axis=1, keepdims=True)
                part = part + jnp.dot(
                    ctx_h.astype(COMM_DT), wo16[h * DH:(h + 1) * DH, :],
                    preferred_element_type=F32)
            acc_ref[pl.ds(b * SQ, SQ), :] = part

        compute_batch(1 - bx)

        seg = p * 0
        rdma0 = None
        for k in range(4):
            half = RS_LEN[k]
            bit = bits[k]
            send_start = seg + (1 - bit) * half
            keep_start = seg + bit * half
            stage_ref[pl.ds(0, half), :] = (
                acc_ref[pl.ds(send_start, half), :].astype(COMM_DT))
            rdma = pltpu.make_async_remote_copy(
                src_ref=stage_ref.at[pl.ds(0, half), :],
                dst_ref=rs_bufs.at[pl.ds(RS_OFF[k], half), :],
                send_sem=send_sems.at[k],
                recv_sem=recv_sems.at[k],
                device_id=(partners[k],),
                device_id_type=MESH_ID,
            )
            rdma.start()
            if k == 0:
                compute_batch(bx)
            rdma.wait()
            sl = pl.ds(keep_start, half)
            acc_ref[sl, :] = acc_ref[sl, :] + (
                rs_bufs[pl.ds(RS_OFF[k], half), :].astype(F32))
            seg = keep_start

        for k in (3, 2, 1, 0):
            L = AG_LEN[k]
            off = AG_OFF[k]
            bit = bits[k]
            partner_start = seg + (1 - 2 * bit) * L
            stage_ref[pl.ds(0, L), :] = (
                acc_ref[pl.ds(seg, L), :].astype(COMM_DT))
            rdma = pltpu.make_async_remote_copy(
                src_ref=stage_ref.at[pl.ds(0, L), :],
                dst_ref=ag_bufs.at[pl.ds(off, L), :],
                send_sem=send_sems.at[7 - k],
                recv_sem=recv_sems.at[7 - k],
                device_id=(partners[k],),
                device_id_type=MESH_ID,
            )
            rdma.start()
            rdma.wait()
            acc_ref[pl.ds(partner_start, L), :] = (
                ag_bufs[pl.ds(off, L), :].astype(F32))
            seg = seg - bit * L

        out_ref[0, :, :] = acc_ref[0:SQ, :]
        out_ref[1, :, :] = acc_ref[SQ:ROWS, :]

    return pl.pallas_call(
        body,
        out_shape=jax.ShapeDtypeStruct((B, SQ, D_MODEL), F32),
        in_specs=[pl.BlockSpec(memory_space=pltpu.VMEM)] * 5,
        out_specs=pl.BlockSpec(memory_space=pltpu.VMEM),
        scratch_shapes=[
            pltpu.VMEM((ROWS, D_MODEL), F32),
            pltpu.VMEM((512, D_MODEL), COMM_DT),
            pltpu.VMEM((960, D_MODEL), COMM_DT),
            pltpu.VMEM((960, D_MODEL), COMM_DT),
            pltpu.SemaphoreType.DMA((8,)),
            pltpu.SemaphoreType.DMA((8,)),
        ],
        compiler_params=pltpu.CompilerParams(collective_id=0),
    )(x, Wq, K_loc, V_loc, Wo)
